# baseline (speedup 1.0000x reference)
import jax
import jax.numpy as jnp
from jax import lax
from jax.experimental import pallas as pl
from jax.experimental.pallas import tpu as pltpu
from jax.experimental.pallas import tpu_sc as plsc

B = 16384
F = 32
NC = 2
NS = 16
L = 16
NW = NC * NS
BPW = B // NW
CHUNK = 128
NCHUNK = BPW // CHUNK
GROUPS = BPW // L


def _gmfd_body(user_r, item_r, user_emb, item_emb, h_w, h_b, out_hbm,
               uidx, iidx, urows, irows, wv, bv, outv, sem):
    wid = lax.axis_index("s") * NC + lax.axis_index("c")
    base = wid * BPW

    pltpu.sync_copy(user_r.at[pl.ds(wid * NCHUNK, NCHUNK)], uidx)
    pltpu.sync_copy(item_r.at[pl.ds(wid * NCHUNK, NCHUNK)], iidx)
    pltpu.sync_copy(h_w, wv)
    pltpu.sync_copy(h_b, bv)

    copies = []
    for j in range(NCHUNK):
        copies.append(pltpu.async_copy(
            user_emb.at[uidx.at[j]], urows.at[pl.ds(j * CHUNK, CHUNK)], sem))
        copies.append(pltpu.async_copy(
            item_emb.at[iidx.at[j]], irows.at[pl.ds(j * CHUNK, CHUNK)], sem))
    for c in copies:
        c.wait()

    lanes = lax.iota(jnp.int32, L)
    w_lo = wv[pl.ds(0, L)]
    w_hi = wv[pl.ds(L, L)]
    bb = bv[...]
    pidx = {h: lanes ^ h for h in (8, 4, 2, 1)}
    hmask = {h: (lanes & h) == 0 for h in (8, 4, 2, 1)}
    brev = [0, 8, 4, 12, 2, 10, 6, 14, 1, 9, 5, 13, 3, 11, 7, 15]

    def gbody(g, _):
        r0 = g * L
        vs = []
        for k in range(L):
            r = r0 + brev[k]
            vs.append(urows[r, pl.ds(0, L)] * irows[r, pl.ds(0, L)] * w_lo
                      + urows[r, pl.ds(L, L)] * irows[r, pl.ds(L, L)] * w_hi)
        for h in (8, 4, 2, 1):
            nxt = []
            for j in range(len(vs) // 2):
                x, y = vs[2 * j], vs[2 * j + 1]
                fx = x + x.at[pidx[h]].get(mode="promise_in_bounds")
                fy = y + y.at[pidx[h]].get(mode="promise_in_bounds")
                nxt.append(jnp.where(hmask[h], fx, fy))
            vs = nxt
        x = vs[0] + bb
        outv[pl.ds(r0, L)] = 1.0 / (1.0 + jnp.exp(-x))
        return 0

    lax.fori_loop(0, GROUPS, gbody, 0)

    pltpu.sync_copy(outv, out_hbm.at[pl.ds(base, BPW)])


def kernel(user, item, user_emb, item_emb, h_w, h_b):
    user_r = user.astype(jnp.int32).reshape(NW * NCHUNK, CHUNK)
    item_r = item.astype(jnp.int32).reshape(NW * NCHUNK, CHUNK)
    w_flat = h_w.reshape(F)
    b_bcast = jnp.broadcast_to(h_b, (L,))
    k = pl.kernel(
        _gmfd_body,
        out_type=jax.ShapeDtypeStruct((B,), jnp.float32),
        mesh=plsc.VectorSubcoreMesh(core_axis_name="c", subcore_axis_name="s"),
        compiler_params=pltpu.CompilerParams(use_tc_tiling_on_sc=False),
        scratch_types=[
            pltpu.VMEM((NCHUNK, CHUNK), jnp.int32),
            pltpu.VMEM((NCHUNK, CHUNK), jnp.int32),
            pltpu.VMEM((BPW, F), jnp.float32),
            pltpu.VMEM((BPW, F), jnp.float32),
            pltpu.VMEM((F,), jnp.float32),
            pltpu.VMEM((L,), jnp.float32),
            pltpu.VMEM((BPW,), jnp.float32),
            pltpu.SemaphoreType.DMA,
        ],
    )
    return k(user_r, item_r, user_emb, item_emb, w_flat, b_bcast)
